# baseline (speedup 1.0000x reference)
import math

import jax
import jax.numpy as jnp
from jax import lax
from jax.experimental import pallas as pl
from jax.experimental.pallas import tpu as pltpu
from jax.experimental.pallas import tpu_sc as plsc

N = 10000
E = 160000
D = 128
NS = 64
NR = 8
G = 64
OUT = 16
R_MAX = 5.0

NSUB = 16
EPT = E // NSUB
BE = 400
NBLK = EPT // BE
ROWS_PT = N // NSUB
ZROWS = 125
CHUNK = 16
NCH = D // CHUNK


def _edge_prep_body(vt_ref, rw0_ref, rw1_ref, orw0, orw1, ounit):
    v = vt_ref[...]
    x2 = jnp.sum(v * v, axis=0, keepdims=True)
    r = jnp.where(x2 == 0.0, 0.0, jnp.where(x2 == 0.0, 1.0, x2) ** 0.5)
    rinv = 1.0 / (r + 1e-9)
    ounit[...] = v * rinv
    nb = (lax.broadcasted_iota(jnp.int32, (NR, 1), 0) + 1).astype(jnp.float32)
    radial = jnp.sin(nb * (math.pi / R_MAX) * r) * rinv
    for rwr, outr in ((rw0_ref, orw0), (rw1_ref, orw1)):
        z = lax.dot_general(radial, rwr[...], (((0,), (0,)), ((), ())),
                            preferred_element_type=jnp.float32)
        outr[...] = z * jax.nn.sigmoid(z)


_BEP = 1280


def _edge_prep(vt, radial_w0, radial_w1):
    return pl.pallas_call(
        _edge_prep_body,
        grid=(E // _BEP,),
        in_specs=[
            pl.BlockSpec((3, _BEP), lambda i: (0, i)),
            pl.BlockSpec((NR, D), lambda i: (0, 0)),
            pl.BlockSpec((NR, D), lambda i: (0, 0)),
        ],
        out_specs=[
            pl.BlockSpec((_BEP, D), lambda i: (i, 0)),
            pl.BlockSpec((_BEP, D), lambda i: (i, 0)),
            pl.BlockSpec((3, _BEP), lambda i: (0, i)),
        ],
        out_shape=[
            jax.ShapeDtypeStruct((E, D), jnp.float32),
            jax.ShapeDtypeStruct((E, D), jnp.float32),
            jax.ShapeDtypeStruct((3, E), jnp.float32),
        ],
    )(vt, radial_w0, radial_w1)


_BN = 2000
_NG = N // _BN


def _init_h_body(sp_ref, emb_ref, oh):
    sp = sp_ref[0]
    ohT = (lax.broadcasted_iota(jnp.int32, (NS, _BN), 0) == sp).astype(jnp.float32)
    oh[...] = lax.dot_general(ohT, emb_ref[...], (((0,), (0,)), ((), ())),
                              preferred_element_type=jnp.float32)


def _init_h(sp3, species_embed):
    return pl.pallas_call(
        _init_h_body,
        grid=(_NG,),
        in_specs=[
            pl.BlockSpec((1, 1, _BN), lambda i: (i, 0, 0)),
            pl.BlockSpec((NS, D), lambda i: (0, 0)),
        ],
        out_specs=pl.BlockSpec((_BN, D), lambda i: (i, 0)),
        out_shape=jax.ShapeDtypeStruct((N, D), jnp.float32),
    )(sp3, species_embed)


def _sc_msg_body(hrows, senders3, receivers3, rw, unitT, agg,
                 sslab, rbuf, hbuf, rwbuf, ubuf, pbuf,
                 ld_sem, st_sem, acc):
    cid = lax.axis_index("c")
    sid = lax.axis_index("s")

    pltpu.sync_copy(senders3.at[sid], sslab)

    @plsc.parallel_loop(0, EPT // 16, unroll=5)
    def _smul(i):
        sslab[pl.ds(i * 16, 16)] = sslab[pl.ds(i * 16, 16)] * NCH

    c0 = jnp.zeros((16,), jnp.int32)
    c1 = jnp.full((16,), 1, jnp.int32)
    c2 = jnp.full((16,), 2, jnp.int32)
    z16 = jnp.zeros((16,), jnp.float32)

    def _chunk(kk, carry):
        chunk = cid * (NCH // 2) + kk

        @plsc.parallel_loop(0, ZROWS, unroll=5)
        def _zrow(i):
            pbuf[0, i, pl.ds(0, 16)] = z16
            pbuf[0, i, pl.ds(16, 16)] = z16
            pbuf[0, i, pl.ds(32, 16)] = z16
            pbuf[0, i, pl.ds(48, 16)] = z16

        zsrc = pbuf.at[0, pl.ds(0, ZROWS)]
        for z in range(ROWS_PT // ZROWS):
            pltpu.sync_copy(
                zsrc, acc.at[pl.ds(sid * ROWS_PT + z * ZROWS, ZROWS)])

        @plsc.parallel_loop(0, EPT // 16, unroll=5)
        def _sadj(i):
            v = sslab[pl.ds(i * 16, 16)]
            sslab[pl.ds(i * 16, 16)] = (v & jnp.int32(-NCH)) + chunk

        plsc.subcore_barrier()

        def _stage(bn):
            nxt = bn % 2
            off = sid * EPT + bn * BE
            return (
                pltpu.async_copy(hrows.at[sslab.at[pl.ds(bn * BE, BE)]],
                                 hbuf.at[nxt], ld_sem),
                pltpu.async_copy(
                    rw.at[pl.ds(off, BE), pl.ds(chunk * CHUNK, CHUNK)],
                    rwbuf.at[nxt], ld_sem),
                pltpu.async_copy(unitT.at[:, pl.ds(off, BE)], ubuf.at[nxt],
                                 ld_sem),
                pltpu.async_copy(receivers3.at[sid, bn], rbuf.at[bn % 3],
                                 ld_sem),
            )

        ld = {0: _stage(0)}
        st = {}
        for b in range(NBLK):
            sel = b % 2
            if b >= 2:
                st.pop(b - 2).wait()
            if b + 1 < NBLK:
                ld[b + 1] = _stage(b + 1)
            for dsc in ld.pop(b):
                dsc.wait()

            ub = ubuf.at[sel]

            @plsc.parallel_loop(0, BE, unroll=16)
            def _edge(e):
                m0 = hbuf[sel, e, pl.ds(0, 16)] * rwbuf[sel, e, pl.ds(0, 16)]
                pbuf[sel, e, pl.ds(0, 16)] = m0
                se = jnp.full((16,), e, jnp.int32)
                ux = plsc.load_gather(ub, [c0, se])
                uy = plsc.load_gather(ub, [c1, se])
                uz = plsc.load_gather(ub, [c2, se])
                pbuf[sel, e, pl.ds(16, 16)] = ux * m0
                pbuf[sel, e, pl.ds(32, 16)] = uy * m0
                pbuf[sel, e, pl.ds(48, 16)] = uz * m0

            st[b] = pltpu.async_copy(pbuf.at[sel], acc.at[rbuf.at[b % 3]],
                                     st_sem, add=True)
        for key in sorted(st):
            st[key].wait()
        plsc.subcore_barrier()
        for a in range(4):
            pltpu.sync_copy(
                acc.at[pl.ds(sid * ROWS_PT, ROWS_PT), pl.ds(a * 16, 16)],
                agg.at[a, pl.ds(sid * ROWS_PT, ROWS_PT),
                       pl.ds(chunk * CHUNK, CHUNK)])
        plsc.subcore_barrier()
        return carry

    lax.fori_loop(0, NCH // 2, _chunk, 0)


def _sc_msg(hrows, senders3, receivers3, rw, unitT):
    mesh = plsc.VectorSubcoreMesh(core_axis_name="c", subcore_axis_name="s")
    fn = pl.kernel(
        _sc_msg_body,
        out_type=jax.ShapeDtypeStruct((4, N, D), jnp.float32),
        mesh=mesh,
        compiler_params=pltpu.CompilerParams(use_tc_tiling_on_sc=False,
                                             needs_layout_passes=False),
        scratch_types=[
            pltpu.VMEM((EPT,), jnp.int32),
            pltpu.VMEM((3, BE), jnp.int32),
            pltpu.VMEM((2, BE, CHUNK), jnp.float32),
            pltpu.VMEM((2, BE, CHUNK), jnp.float32),
            pltpu.VMEM((2, 3, BE), jnp.float32),
            pltpu.VMEM((2, BE, 4 * CHUNK), jnp.float32),
            pltpu.SemaphoreType.DMA,
            pltpu.SemaphoreType.DMA,
            pltpu.VMEM_SHARED((N, 4 * CHUNK), jnp.float32),
        ],
    )
    return fn(hrows, senders3, receivers3, rw, unitT)


def _node_upd_body(agg_ref, h_ref, sp_ref, mw_ref, sw_ref, sb_ref, ls_ref,
                   oh):
    h = h_ref[...]
    s = agg_ref[0]
    vx = agg_ref[1]
    vy = agg_ref[2]
    vz = agg_ref[3]
    q = vx * vx + vy * vy + vz * vz
    inv = jnp.where(q == 0.0, 0.0, jnp.where(q == 0.0, 1.0, q) ** 0.5)
    sp = sp_ref[0]
    ohT = (lax.broadcasted_iota(jnp.int32, (NS, _BN), 0) == sp).astype(jnp.float32)
    bias = lax.dot_general(ohT, sb_ref[...], (((0,), (0,)), ((), ())),
                           preferred_element_type=jnp.float32)
    x = (jnp.dot(s + inv, mw_ref[...], preferred_element_type=jnp.float32)
         + jnp.dot(h, sw_ref[...], preferred_element_type=jnp.float32)
         + bias)
    mu = jnp.mean(x, axis=-1, keepdims=True)
    var = jnp.mean((x - mu) * (x - mu), axis=-1, keepdims=True)
    y = (x - mu) / jnp.sqrt(var + 1e-6) * ls_ref[...]
    oh[...] = y + h


def _node_upd(agg, h, sp3, mw, sw, sb, ls2):
    return pl.pallas_call(
        _node_upd_body,
        grid=(_NG,),
        in_specs=[
            pl.BlockSpec((4, _BN, D), lambda i: (0, i, 0)),
            pl.BlockSpec((_BN, D), lambda i: (i, 0)),
            pl.BlockSpec((1, 1, _BN), lambda i: (i, 0, 0)),
            pl.BlockSpec((D, D), lambda i: (0, 0)),
            pl.BlockSpec((D, D), lambda i: (0, 0)),
            pl.BlockSpec((NS, D), lambda i: (0, 0)),
            pl.BlockSpec((1, D), lambda i: (0, 0)),
        ],
        out_specs=pl.BlockSpec((_BN, D), lambda i: (i, 0)),
        out_shape=jax.ShapeDtypeStruct((N, D), jnp.float32),
    )(agg, h, sp3, mw, sw, sb, ls2)


def _readout_body(h_ref, gid_ref, rw_ref, w1_ref, w2_ref, sc_ref, sh_ref,
                  out_ref, acc_ref, cnt_ref):
    i = pl.program_id(0)

    @pl.when(i == 0)
    def _():
        acc_ref[...] = jnp.zeros_like(acc_ref)
        cnt_ref[...] = jnp.zeros_like(cnt_ref)

    h = h_ref[...]
    o = jnp.dot(h, rw_ref[...], preferred_element_type=jnp.float32)
    mu = jnp.mean(o, axis=-1, keepdims=True)
    var = jnp.mean((o - mu) * (o - mu), axis=-1, keepdims=True)
    o = (o - mu) / jnp.sqrt(var + 1e-6)
    hid = jnp.dot(o, w1_ref[...], preferred_element_type=jnp.float32)
    hid = hid * jax.nn.sigmoid(hid)
    npred = jnp.dot(hid, w2_ref[...], preferred_element_type=jnp.float32)
    ohT = (lax.broadcasted_iota(jnp.int32, (G, _BN), 0) == gid_ref[0]).astype(jnp.float32)
    acc_ref[...] += lax.dot_general(ohT, npred, (((1,), (0,)), ((), ())),
                                    preferred_element_type=jnp.float32)
    cnt_ref[...] += jnp.sum(ohT, axis=1, keepdims=True)

    @pl.when(i == _NG - 1)
    def _():
        gm = acc_ref[...] / jnp.maximum(cnt_ref[...], 1.0)
        out_ref[...] = gm * sc_ref[0] + sh_ref[0]


def _readout(h, gid3, readout_w, head_w1, head_w2, out_scale, out_shift):
    return pl.pallas_call(
        _readout_body,
        grid=(_NG,),
        in_specs=[
            pl.BlockSpec((_BN, D), lambda i: (i, 0)),
            pl.BlockSpec((1, 1, _BN), lambda i: (i, 0, 0)),
            pl.BlockSpec((D, OUT), lambda i: (0, 0)),
            pl.BlockSpec((OUT, 64), lambda i: (0, 0)),
            pl.BlockSpec((64, 1), lambda i: (0, 0)),
            pl.BlockSpec(memory_space=pltpu.SMEM),
            pl.BlockSpec(memory_space=pltpu.SMEM),
        ],
        out_specs=pl.BlockSpec((G, 1), lambda i: (0, 0)),
        out_shape=jax.ShapeDtypeStruct((G, 1), jnp.float32),
        scratch_shapes=[
            pltpu.VMEM((G, 1), jnp.float32),
            pltpu.VMEM((G, 1), jnp.float32),
        ],
    )(h, gid3, readout_w, head_w1, head_w2, out_scale, out_shift)


def kernel(vectors, node_species, receivers, senders, graph_ids,
           species_embed, radial_w0, msg_w0, self_w0, species_b0, ln_s0,
           radial_w1, msg_w1, self_w1, species_b1, ln_s1, readout_w,
           head_w1, head_w2, out_scale, out_shift):
    vt = vectors.T
    senders3 = senders.reshape(NSUB, EPT)
    receivers3 = receivers.reshape(NSUB, NBLK, BE)
    rw0, rw1, unitT = _edge_prep(vt, radial_w0, radial_w1)
    sp3 = node_species.reshape(_NG, 1, _BN)
    h = _init_h(sp3, species_embed)
    for (rw, mw, sw, sb, ls) in ((rw0, msg_w0, self_w0, species_b0, ln_s0),
                                 (rw1, msg_w1, self_w1, species_b1, ln_s1)):
        agg = _sc_msg(h.reshape(NCH * N, CHUNK), senders3, receivers3, rw, unitT)
        h = _node_upd(agg, h, sp3, mw, sw, sb, ls.reshape(1, D))
    gid3 = graph_ids.reshape(_NG, 1, _BN)
    return _readout(h, gid3, readout_w, head_w1, head_w2,
                    out_scale.reshape(1), out_shift.reshape(1))


# speedup vs baseline: 1.0448x; 1.0448x over previous
import math

import jax
import jax.numpy as jnp
from jax import lax
from jax.experimental import pallas as pl
from jax.experimental.pallas import tpu as pltpu
from jax.experimental.pallas import tpu_sc as plsc

N = 10000
E = 160000
D = 128
NS = 64
NR = 8
G = 64
OUT = 16
R_MAX = 5.0

NSUB = 16
EPT = E // NSUB
BE = 400
NBLK = EPT // BE
ROWS_PT = N // NSUB
ZROWS = 125
CHUNK = 16
NCH = D // CHUNK


def _edge_prep_body(vt_ref, rw0_ref, orw0, ounit):
    v = vt_ref[...]
    x2 = jnp.sum(v * v, axis=0, keepdims=True)
    r = jnp.where(x2 == 0.0, 0.0, jnp.where(x2 == 0.0, 1.0, x2) ** 0.5)
    rinv = 1.0 / (r + 1e-9)
    ounit[...] = v * rinv
    nb = (lax.broadcasted_iota(jnp.int32, (NR, 1), 0) + 1).astype(jnp.float32)
    radial = jnp.sin(nb * (math.pi / R_MAX) * r) * rinv
    z = lax.dot_general(radial, rw0_ref[...], (((0,), (0,)), ((), ())),
                        preferred_element_type=jnp.float32)
    orw0[...] = z * jax.nn.sigmoid(z)


def _edge_prep2_body(vt_ref, rw1_ref, orw1):
    v = vt_ref[...]
    x2 = jnp.sum(v * v, axis=0, keepdims=True)
    r = jnp.where(x2 == 0.0, 0.0, jnp.where(x2 == 0.0, 1.0, x2) ** 0.5)
    rinv = 1.0 / (r + 1e-9)
    nb = (lax.broadcasted_iota(jnp.int32, (NR, 1), 0) + 1).astype(jnp.float32)
    radial = jnp.sin(nb * (math.pi / R_MAX) * r) * rinv
    z = lax.dot_general(radial, rw1_ref[...], (((0,), (0,)), ((), ())),
                        preferred_element_type=jnp.float32)
    orw1[...] = z * jax.nn.sigmoid(z)


_BEP = 1280


def _edge_prep(vt, radial_w0):
    return pl.pallas_call(
        _edge_prep_body,
        grid=(E // _BEP,),
        in_specs=[
            pl.BlockSpec((3, _BEP), lambda i: (0, i)),
            pl.BlockSpec((NR, D), lambda i: (0, 0)),
        ],
        out_specs=[
            pl.BlockSpec((_BEP, D), lambda i: (i, 0)),
            pl.BlockSpec((3, _BEP), lambda i: (0, i)),
        ],
        out_shape=[
            jax.ShapeDtypeStruct((E, D), jnp.float32),
            jax.ShapeDtypeStruct((3, E), jnp.float32),
        ],
    )(vt, radial_w0)


def _edge_prep2(vt, radial_w1):
    return pl.pallas_call(
        _edge_prep2_body,
        grid=(E // _BEP,),
        in_specs=[
            pl.BlockSpec((3, _BEP), lambda i: (0, i)),
            pl.BlockSpec((NR, D), lambda i: (0, 0)),
        ],
        out_specs=pl.BlockSpec((_BEP, D), lambda i: (i, 0)),
        out_shape=jax.ShapeDtypeStruct((E, D), jnp.float32),
    )(vt, radial_w1)


_BN = 2000
_NG = N // _BN


def _init_h_body(sp_ref, emb_ref, oh):
    sp = sp_ref[0]
    ohT = (lax.broadcasted_iota(jnp.int32, (NS, _BN), 0) == sp).astype(jnp.float32)
    oh[...] = lax.dot_general(ohT, emb_ref[...], (((0,), (0,)), ((), ())),
                              preferred_element_type=jnp.float32)


def _init_h(sp3, species_embed):
    return pl.pallas_call(
        _init_h_body,
        grid=(_NG,),
        in_specs=[
            pl.BlockSpec((1, 1, _BN), lambda i: (i, 0, 0)),
            pl.BlockSpec((NS, D), lambda i: (0, 0)),
        ],
        out_specs=pl.BlockSpec((_BN, D), lambda i: (i, 0)),
        out_shape=jax.ShapeDtypeStruct((N, D), jnp.float32),
    )(sp3, species_embed)


def _sc_msg_body(hrows, senders3, receivers3, rw, unitT, agg,
                 sslab, rbuf, hbuf, rwbuf, ubuf, pbuf,
                 ld_sem, st_sem, acc):
    cid = lax.axis_index("c")
    sid = lax.axis_index("s")

    pltpu.sync_copy(senders3.at[sid], sslab)

    @plsc.parallel_loop(0, EPT // 16, unroll=5)
    def _smul(i):
        sslab[pl.ds(i * 16, 16)] = sslab[pl.ds(i * 16, 16)] * NCH

    c0 = jnp.zeros((16,), jnp.int32)
    c1 = jnp.full((16,), 1, jnp.int32)
    c2 = jnp.full((16,), 2, jnp.int32)
    z16 = jnp.zeros((16,), jnp.float32)

    def _chunk(kk, carry):
        chunk = cid * (NCH // 2) + kk

        @plsc.parallel_loop(0, ZROWS, unroll=5)
        def _zrow(i):
            pbuf[0, i, pl.ds(0, 16)] = z16
            pbuf[0, i, pl.ds(16, 16)] = z16
            pbuf[0, i, pl.ds(32, 16)] = z16
            pbuf[0, i, pl.ds(48, 16)] = z16

        zsrc = pbuf.at[0, pl.ds(0, ZROWS)]
        for z in range(ROWS_PT // ZROWS):
            pltpu.sync_copy(
                zsrc, acc.at[pl.ds(sid * ROWS_PT + z * ZROWS, ZROWS)])

        @plsc.parallel_loop(0, EPT // 16, unroll=5)
        def _sadj(i):
            v = sslab[pl.ds(i * 16, 16)]
            sslab[pl.ds(i * 16, 16)] = (v & jnp.int32(-NCH)) + chunk

        plsc.subcore_barrier()

        def _stage(bn):
            nxt = bn % 2
            off = sid * EPT + bn * BE
            return (
                pltpu.async_copy(hrows.at[sslab.at[pl.ds(bn * BE, BE)]],
                                 hbuf.at[nxt], ld_sem),
                pltpu.async_copy(
                    rw.at[pl.ds(off, BE), pl.ds(chunk * CHUNK, CHUNK)],
                    rwbuf.at[nxt], ld_sem),
                pltpu.async_copy(unitT.at[:, pl.ds(off, BE)], ubuf.at[nxt],
                                 ld_sem),
                pltpu.async_copy(receivers3.at[sid, bn], rbuf.at[bn % 3],
                                 ld_sem),
            )

        ld = {0: _stage(0)}
        st = {}
        for b in range(NBLK):
            sel = b % 2
            if b >= 2:
                st.pop(b - 2).wait()
            if b + 1 < NBLK:
                ld[b + 1] = _stage(b + 1)
            for dsc in ld.pop(b):
                dsc.wait()

            ub = ubuf.at[sel]

            @plsc.parallel_loop(0, BE, unroll=8)
            def _edge(e):
                m0 = hbuf[sel, e, pl.ds(0, 16)] * rwbuf[sel, e, pl.ds(0, 16)]
                pbuf[sel, e, pl.ds(0, 16)] = m0
                se = jnp.full((16,), e, jnp.int32)
                ux = plsc.load_gather(ub, [c0, se])
                uy = plsc.load_gather(ub, [c1, se])
                uz = plsc.load_gather(ub, [c2, se])
                pbuf[sel, e, pl.ds(16, 16)] = ux * m0
                pbuf[sel, e, pl.ds(32, 16)] = uy * m0
                pbuf[sel, e, pl.ds(48, 16)] = uz * m0

            st[b] = pltpu.async_copy(pbuf.at[sel], acc.at[rbuf.at[b % 3]],
                                     st_sem, add=True)
        for key in sorted(st):
            st[key].wait()
        plsc.subcore_barrier()
        for a in range(4):
            pltpu.sync_copy(
                acc.at[pl.ds(sid * ROWS_PT, ROWS_PT), pl.ds(a * 16, 16)],
                agg.at[a, pl.ds(sid * ROWS_PT, ROWS_PT),
                       pl.ds(chunk * CHUNK, CHUNK)])
        plsc.subcore_barrier()
        return carry

    lax.fori_loop(0, NCH // 2, _chunk, 0)


def _sc_msg(hrows, senders3, receivers3, rw, unitT):
    mesh = plsc.VectorSubcoreMesh(core_axis_name="c", subcore_axis_name="s")
    fn = pl.kernel(
        _sc_msg_body,
        out_type=jax.ShapeDtypeStruct((4, N, D), jnp.float32),
        mesh=mesh,
        compiler_params=pltpu.CompilerParams(use_tc_tiling_on_sc=False,
                                             needs_layout_passes=False),
        scratch_types=[
            pltpu.VMEM((EPT,), jnp.int32),
            pltpu.VMEM((3, BE), jnp.int32),
            pltpu.VMEM((2, BE, CHUNK), jnp.float32),
            pltpu.VMEM((2, BE, CHUNK), jnp.float32),
            pltpu.VMEM((2, 3, BE), jnp.float32),
            pltpu.VMEM((2, BE, 4 * CHUNK), jnp.float32),
            pltpu.SemaphoreType.DMA,
            pltpu.SemaphoreType.DMA,
            pltpu.VMEM_SHARED((N, 4 * CHUNK), jnp.float32),
        ],
    )
    return fn(hrows, senders3, receivers3, rw, unitT)


def _node_upd_body(agg_ref, h_ref, sp_ref, mw_ref, sw_ref, sb_ref, ls_ref,
                   oh):
    h = h_ref[...]
    s = agg_ref[0]
    vx = agg_ref[1]
    vy = agg_ref[2]
    vz = agg_ref[3]
    q = vx * vx + vy * vy + vz * vz
    inv = jnp.where(q == 0.0, 0.0, jnp.where(q == 0.0, 1.0, q) ** 0.5)
    sp = sp_ref[0]
    ohT = (lax.broadcasted_iota(jnp.int32, (NS, _BN), 0) == sp).astype(jnp.float32)
    bias = lax.dot_general(ohT, sb_ref[...], (((0,), (0,)), ((), ())),
                           preferred_element_type=jnp.float32)
    x = (jnp.dot(s + inv, mw_ref[...], preferred_element_type=jnp.float32)
         + jnp.dot(h, sw_ref[...], preferred_element_type=jnp.float32)
         + bias)
    mu = jnp.mean(x, axis=-1, keepdims=True)
    var = jnp.mean((x - mu) * (x - mu), axis=-1, keepdims=True)
    y = (x - mu) / jnp.sqrt(var + 1e-6) * ls_ref[...]
    oh[...] = y + h


def _node_upd(agg, h, sp3, mw, sw, sb, ls2):
    return pl.pallas_call(
        _node_upd_body,
        grid=(_NG,),
        in_specs=[
            pl.BlockSpec((4, _BN, D), lambda i: (0, i, 0)),
            pl.BlockSpec((_BN, D), lambda i: (i, 0)),
            pl.BlockSpec((1, 1, _BN), lambda i: (i, 0, 0)),
            pl.BlockSpec((D, D), lambda i: (0, 0)),
            pl.BlockSpec((D, D), lambda i: (0, 0)),
            pl.BlockSpec((NS, D), lambda i: (0, 0)),
            pl.BlockSpec((1, D), lambda i: (0, 0)),
        ],
        out_specs=pl.BlockSpec((_BN, D), lambda i: (i, 0)),
        out_shape=jax.ShapeDtypeStruct((N, D), jnp.float32),
    )(agg, h, sp3, mw, sw, sb, ls2)


def _readout_body(h_ref, gid_ref, rw_ref, w1_ref, w2_ref, sc_ref, sh_ref,
                  out_ref, acc_ref, cnt_ref):
    i = pl.program_id(0)

    @pl.when(i == 0)
    def _():
        acc_ref[...] = jnp.zeros_like(acc_ref)
        cnt_ref[...] = jnp.zeros_like(cnt_ref)

    h = h_ref[...]
    o = jnp.dot(h, rw_ref[...], preferred_element_type=jnp.float32)
    mu = jnp.mean(o, axis=-1, keepdims=True)
    var = jnp.mean((o - mu) * (o - mu), axis=-1, keepdims=True)
    o = (o - mu) / jnp.sqrt(var + 1e-6)
    hid = jnp.dot(o, w1_ref[...], preferred_element_type=jnp.float32)
    hid = hid * jax.nn.sigmoid(hid)
    npred = jnp.dot(hid, w2_ref[...], preferred_element_type=jnp.float32)
    ohT = (lax.broadcasted_iota(jnp.int32, (G, _BN), 0) == gid_ref[0]).astype(jnp.float32)
    acc_ref[...] += lax.dot_general(ohT, npred, (((1,), (0,)), ((), ())),
                                    preferred_element_type=jnp.float32)
    cnt_ref[...] += jnp.sum(ohT, axis=1, keepdims=True)

    @pl.when(i == _NG - 1)
    def _():
        gm = acc_ref[...] / jnp.maximum(cnt_ref[...], 1.0)
        out_ref[...] = gm * sc_ref[0] + sh_ref[0]


def _readout(h, gid3, readout_w, head_w1, head_w2, out_scale, out_shift):
    return pl.pallas_call(
        _readout_body,
        grid=(_NG,),
        in_specs=[
            pl.BlockSpec((_BN, D), lambda i: (i, 0)),
            pl.BlockSpec((1, 1, _BN), lambda i: (i, 0, 0)),
            pl.BlockSpec((D, OUT), lambda i: (0, 0)),
            pl.BlockSpec((OUT, 64), lambda i: (0, 0)),
            pl.BlockSpec((64, 1), lambda i: (0, 0)),
            pl.BlockSpec(memory_space=pltpu.SMEM),
            pl.BlockSpec(memory_space=pltpu.SMEM),
        ],
        out_specs=pl.BlockSpec((G, 1), lambda i: (0, 0)),
        out_shape=jax.ShapeDtypeStruct((G, 1), jnp.float32),
        scratch_shapes=[
            pltpu.VMEM((G, 1), jnp.float32),
            pltpu.VMEM((G, 1), jnp.float32),
        ],
    )(h, gid3, readout_w, head_w1, head_w2, out_scale, out_shift)


def kernel(vectors, node_species, receivers, senders, graph_ids,
           species_embed, radial_w0, msg_w0, self_w0, species_b0, ln_s0,
           radial_w1, msg_w1, self_w1, species_b1, ln_s1, readout_w,
           head_w1, head_w2, out_scale, out_shift):
    vt = vectors.T
    senders3 = senders.reshape(NSUB, EPT)
    receivers3 = receivers.reshape(NSUB, NBLK, BE)
    rw0, unitT = _edge_prep(vt, radial_w0)
    sp3 = node_species.reshape(_NG, 1, _BN)
    h = _init_h(sp3, species_embed)
    agg = _sc_msg(h.reshape(NCH * N, CHUNK), senders3, receivers3, rw0, unitT)
    rw1 = _edge_prep2(vt, radial_w1)
    h = _node_upd(agg, h, sp3, msg_w0, self_w0, species_b0, ln_s0.reshape(1, D))
    agg = _sc_msg(h.reshape(NCH * N, CHUNK), senders3, receivers3, rw1, unitT)
    h = _node_upd(agg, h, sp3, msg_w1, self_w1, species_b1, ln_s1.reshape(1, D))
    gid3 = graph_ids.reshape(_NG, 1, _BN)
    return _readout(h, gid3, readout_w, head_w1, head_w2,
                    out_scale.reshape(1), out_shift.reshape(1))


# speedup vs baseline: 1.0522x; 1.0070x over previous
import math

import jax
import jax.numpy as jnp
from jax import lax
from jax.experimental import pallas as pl
from jax.experimental.pallas import tpu as pltpu
from jax.experimental.pallas import tpu_sc as plsc

N = 10000
E = 160000
D = 128
NS = 64
NR = 8
G = 64
OUT = 16
R_MAX = 5.0

NSUB = 16
EPT = E // NSUB
BE = 400
NBLK = EPT // BE
ROWS_PT = N // NSUB
ZROWS = 125
CHUNK = 16
NCH = D // CHUNK


def _edge_prep_body(vt_ref, rw0_ref, orw0, ounit):
    v = vt_ref[...]
    x2 = jnp.sum(v * v, axis=0, keepdims=True)
    r = jnp.where(x2 == 0.0, 0.0, jnp.where(x2 == 0.0, 1.0, x2) ** 0.5)
    rinv = 1.0 / (r + 1e-9)
    ounit[...] = v * rinv
    nb = (lax.broadcasted_iota(jnp.int32, (NR, 1), 0) + 1).astype(jnp.float32)
    radial = jnp.sin(nb * (math.pi / R_MAX) * r) * rinv
    z = lax.dot_general(radial, rw0_ref[...], (((0,), (0,)), ((), ())),
                        preferred_element_type=jnp.float32)
    orw0[...] = z * jax.nn.sigmoid(z)


def _edge_prep2_body(vt_ref, rw1_ref, orw1):
    v = vt_ref[...]
    x2 = jnp.sum(v * v, axis=0, keepdims=True)
    r = jnp.where(x2 == 0.0, 0.0, jnp.where(x2 == 0.0, 1.0, x2) ** 0.5)
    rinv = 1.0 / (r + 1e-9)
    nb = (lax.broadcasted_iota(jnp.int32, (NR, 1), 0) + 1).astype(jnp.float32)
    radial = jnp.sin(nb * (math.pi / R_MAX) * r) * rinv
    z = lax.dot_general(radial, rw1_ref[...], (((0,), (0,)), ((), ())),
                        preferred_element_type=jnp.float32)
    orw1[...] = z * jax.nn.sigmoid(z)


_BEP = 1280


def _edge_prep(vt, radial_w0):
    return pl.pallas_call(
        _edge_prep_body,
        grid=(E // _BEP,),
        in_specs=[
            pl.BlockSpec((3, _BEP), lambda i: (0, i)),
            pl.BlockSpec((NR, D), lambda i: (0, 0)),
        ],
        out_specs=[
            pl.BlockSpec((_BEP, D), lambda i: (i, 0)),
            pl.BlockSpec((3, _BEP), lambda i: (0, i)),
        ],
        out_shape=[
            jax.ShapeDtypeStruct((E, D), jnp.float32),
            jax.ShapeDtypeStruct((3, E), jnp.float32),
        ],
    )(vt, radial_w0)


def _edge_prep2(vt, radial_w1):
    return pl.pallas_call(
        _edge_prep2_body,
        grid=(E // _BEP,),
        in_specs=[
            pl.BlockSpec((3, _BEP), lambda i: (0, i)),
            pl.BlockSpec((NR, D), lambda i: (0, 0)),
        ],
        out_specs=pl.BlockSpec((_BEP, D), lambda i: (i, 0)),
        out_shape=jax.ShapeDtypeStruct((E, D), jnp.float32),
    )(vt, radial_w1)


_BN = 2000
_NG = N // _BN


def _init_h_body(sp_ref, emb_ref, oh):
    sp = sp_ref[0]
    ohT = (lax.broadcasted_iota(jnp.int32, (NS, _BN), 0) == sp).astype(jnp.float32)
    oh[...] = lax.dot_general(ohT, emb_ref[...], (((0,), (0,)), ((), ())),
                              preferred_element_type=jnp.float32)


def _init_h(sp3, species_embed):
    return pl.pallas_call(
        _init_h_body,
        grid=(_NG,),
        in_specs=[
            pl.BlockSpec((1, 1, _BN), lambda i: (i, 0, 0)),
            pl.BlockSpec((NS, D), lambda i: (0, 0)),
        ],
        out_specs=pl.BlockSpec((_BN, D), lambda i: (i, 0)),
        out_shape=jax.ShapeDtypeStruct((N, D), jnp.float32),
    )(sp3, species_embed)


def _sc_msg_body(hrows, senders3, receivers3, rw, unitT, agg,
                 sslab, rbuf, hbuf, rwbuf, ubuf, pbuf,
                 ld_sem, st_sem, acc):
    cid = lax.axis_index("c")
    sid = lax.axis_index("s")

    pltpu.sync_copy(senders3.at[sid], sslab)

    @plsc.parallel_loop(0, EPT // 16, unroll=5)
    def _smul(i):
        sslab[pl.ds(i * 16, 16)] = sslab[pl.ds(i * 16, 16)] * NCH

    c0 = jnp.zeros((16,), jnp.int32)
    c1 = jnp.full((16,), 1, jnp.int32)
    c2 = jnp.full((16,), 2, jnp.int32)
    z16 = jnp.zeros((16,), jnp.float32)

    def _chunk(kk, carry):
        chunk = cid * (NCH // 2) + kk

        @plsc.parallel_loop(0, ZROWS, unroll=5)
        def _zrow(i):
            pbuf[0, i, pl.ds(0, 16)] = z16
            pbuf[0, i, pl.ds(16, 16)] = z16
            pbuf[0, i, pl.ds(32, 16)] = z16
            pbuf[0, i, pl.ds(48, 16)] = z16

        zsrc = pbuf.at[0, pl.ds(0, ZROWS)]
        zd = [pltpu.async_copy(
                  zsrc, acc.at[pl.ds(sid * ROWS_PT + z * ZROWS, ZROWS)],
                  ld_sem)
              for z in range(ROWS_PT // ZROWS)]
        for dsc in zd:
            dsc.wait()

        @plsc.parallel_loop(0, EPT // 16, unroll=5)
        def _sadj(i):
            v = sslab[pl.ds(i * 16, 16)]
            sslab[pl.ds(i * 16, 16)] = (v & jnp.int32(-NCH)) + chunk

        plsc.subcore_barrier()

        def _stage(bn):
            nxt = bn % 2
            off = sid * EPT + bn * BE
            return (
                pltpu.async_copy(hrows.at[sslab.at[pl.ds(bn * BE, BE)]],
                                 hbuf.at[nxt], ld_sem),
                pltpu.async_copy(
                    rw.at[pl.ds(off, BE), pl.ds(chunk * CHUNK, CHUNK)],
                    rwbuf.at[nxt], ld_sem),
                pltpu.async_copy(unitT.at[:, pl.ds(off, BE)], ubuf.at[nxt],
                                 ld_sem),
                pltpu.async_copy(receivers3.at[sid, bn], rbuf.at[bn % 3],
                                 ld_sem),
            )

        ld = {0: _stage(0)}
        st = {}
        for b in range(NBLK):
            sel = b % 2
            if b >= 2:
                st.pop(b - 2).wait()
            if b + 1 < NBLK:
                ld[b + 1] = _stage(b + 1)
            for dsc in ld.pop(b):
                dsc.wait()

            ub = ubuf.at[sel]

            @plsc.parallel_loop(0, BE, unroll=8)
            def _edge(e):
                m0 = hbuf[sel, e, pl.ds(0, 16)] * rwbuf[sel, e, pl.ds(0, 16)]
                pbuf[sel, e, pl.ds(0, 16)] = m0
                se = jnp.full((16,), e, jnp.int32)
                ux = plsc.load_gather(ub, [c0, se])
                uy = plsc.load_gather(ub, [c1, se])
                uz = plsc.load_gather(ub, [c2, se])
                pbuf[sel, e, pl.ds(16, 16)] = ux * m0
                pbuf[sel, e, pl.ds(32, 16)] = uy * m0
                pbuf[sel, e, pl.ds(48, 16)] = uz * m0

            st[b] = pltpu.async_copy(pbuf.at[sel], acc.at[rbuf.at[b % 3]],
                                     st_sem, add=True)
        for key in sorted(st):
            st[key].wait()
        plsc.subcore_barrier()
        od = [pltpu.async_copy(
                  acc.at[pl.ds(sid * ROWS_PT, ROWS_PT), pl.ds(a * 16, 16)],
                  agg.at[a, pl.ds(sid * ROWS_PT, ROWS_PT),
                         pl.ds(chunk * CHUNK, CHUNK)], ld_sem)
              for a in range(4)]
        for dsc in od:
            dsc.wait()
        plsc.subcore_barrier()
        return carry

    lax.fori_loop(0, NCH // 2, _chunk, 0)


def _sc_msg(hrows, senders3, receivers3, rw, unitT):
    mesh = plsc.VectorSubcoreMesh(core_axis_name="c", subcore_axis_name="s")
    fn = pl.kernel(
        _sc_msg_body,
        out_type=jax.ShapeDtypeStruct((4, N, D), jnp.float32),
        mesh=mesh,
        compiler_params=pltpu.CompilerParams(use_tc_tiling_on_sc=False,
                                             needs_layout_passes=False),
        scratch_types=[
            pltpu.VMEM((EPT,), jnp.int32),
            pltpu.VMEM((3, BE), jnp.int32),
            pltpu.VMEM((2, BE, CHUNK), jnp.float32),
            pltpu.VMEM((2, BE, CHUNK), jnp.float32),
            pltpu.VMEM((2, 3, BE), jnp.float32),
            pltpu.VMEM((2, BE, 4 * CHUNK), jnp.float32),
            pltpu.SemaphoreType.DMA,
            pltpu.SemaphoreType.DMA,
            pltpu.VMEM_SHARED((N, 4 * CHUNK), jnp.float32),
        ],
    )
    return fn(hrows, senders3, receivers3, rw, unitT)


def _node_upd_body(agg_ref, h_ref, sp_ref, mw_ref, sw_ref, sb_ref, ls_ref,
                   oh):
    h = h_ref[...]
    s = agg_ref[0]
    vx = agg_ref[1]
    vy = agg_ref[2]
    vz = agg_ref[3]
    q = vx * vx + vy * vy + vz * vz
    inv = jnp.where(q == 0.0, 0.0, jnp.where(q == 0.0, 1.0, q) ** 0.5)
    sp = sp_ref[0]
    ohT = (lax.broadcasted_iota(jnp.int32, (NS, _BN), 0) == sp).astype(jnp.float32)
    bias = lax.dot_general(ohT, sb_ref[...], (((0,), (0,)), ((), ())),
                           preferred_element_type=jnp.float32)
    x = (jnp.dot(s + inv, mw_ref[...], preferred_element_type=jnp.float32)
         + jnp.dot(h, sw_ref[...], preferred_element_type=jnp.float32)
         + bias)
    mu = jnp.mean(x, axis=-1, keepdims=True)
    var = jnp.mean((x - mu) * (x - mu), axis=-1, keepdims=True)
    y = (x - mu) / jnp.sqrt(var + 1e-6) * ls_ref[...]
    oh[...] = y + h


def _node_upd(agg, h, sp3, mw, sw, sb, ls2):
    return pl.pallas_call(
        _node_upd_body,
        grid=(_NG,),
        in_specs=[
            pl.BlockSpec((4, _BN, D), lambda i: (0, i, 0)),
            pl.BlockSpec((_BN, D), lambda i: (i, 0)),
            pl.BlockSpec((1, 1, _BN), lambda i: (i, 0, 0)),
            pl.BlockSpec((D, D), lambda i: (0, 0)),
            pl.BlockSpec((D, D), lambda i: (0, 0)),
            pl.BlockSpec((NS, D), lambda i: (0, 0)),
            pl.BlockSpec((1, D), lambda i: (0, 0)),
        ],
        out_specs=pl.BlockSpec((_BN, D), lambda i: (i, 0)),
        out_shape=jax.ShapeDtypeStruct((N, D), jnp.float32),
    )(agg, h, sp3, mw, sw, sb, ls2)


def _readout_body(h_ref, gid_ref, rw_ref, w1_ref, w2_ref, sc_ref, sh_ref,
                  out_ref, acc_ref, cnt_ref):
    i = pl.program_id(0)

    @pl.when(i == 0)
    def _():
        acc_ref[...] = jnp.zeros_like(acc_ref)
        cnt_ref[...] = jnp.zeros_like(cnt_ref)

    h = h_ref[...]
    o = jnp.dot(h, rw_ref[...], preferred_element_type=jnp.float32)
    mu = jnp.mean(o, axis=-1, keepdims=True)
    var = jnp.mean((o - mu) * (o - mu), axis=-1, keepdims=True)
    o = (o - mu) / jnp.sqrt(var + 1e-6)
    hid = jnp.dot(o, w1_ref[...], preferred_element_type=jnp.float32)
    hid = hid * jax.nn.sigmoid(hid)
    npred = jnp.dot(hid, w2_ref[...], preferred_element_type=jnp.float32)
    ohT = (lax.broadcasted_iota(jnp.int32, (G, _BN), 0) == gid_ref[0]).astype(jnp.float32)
    acc_ref[...] += lax.dot_general(ohT, npred, (((1,), (0,)), ((), ())),
                                    preferred_element_type=jnp.float32)
    cnt_ref[...] += jnp.sum(ohT, axis=1, keepdims=True)

    @pl.when(i == _NG - 1)
    def _():
        gm = acc_ref[...] / jnp.maximum(cnt_ref[...], 1.0)
        out_ref[...] = gm * sc_ref[0] + sh_ref[0]


def _readout(h, gid3, readout_w, head_w1, head_w2, out_scale, out_shift):
    return pl.pallas_call(
        _readout_body,
        grid=(_NG,),
        in_specs=[
            pl.BlockSpec((_BN, D), lambda i: (i, 0)),
            pl.BlockSpec((1, 1, _BN), lambda i: (i, 0, 0)),
            pl.BlockSpec((D, OUT), lambda i: (0, 0)),
            pl.BlockSpec((OUT, 64), lambda i: (0, 0)),
            pl.BlockSpec((64, 1), lambda i: (0, 0)),
            pl.BlockSpec(memory_space=pltpu.SMEM),
            pl.BlockSpec(memory_space=pltpu.SMEM),
        ],
        out_specs=pl.BlockSpec((G, 1), lambda i: (0, 0)),
        out_shape=jax.ShapeDtypeStruct((G, 1), jnp.float32),
        scratch_shapes=[
            pltpu.VMEM((G, 1), jnp.float32),
            pltpu.VMEM((G, 1), jnp.float32),
        ],
    )(h, gid3, readout_w, head_w1, head_w2, out_scale, out_shift)


def kernel(vectors, node_species, receivers, senders, graph_ids,
           species_embed, radial_w0, msg_w0, self_w0, species_b0, ln_s0,
           radial_w1, msg_w1, self_w1, species_b1, ln_s1, readout_w,
           head_w1, head_w2, out_scale, out_shift):
    vt = vectors.T
    senders3 = senders.reshape(NSUB, EPT)
    receivers3 = receivers.reshape(NSUB, NBLK, BE)
    rw0, unitT = _edge_prep(vt, radial_w0)
    sp3 = node_species.reshape(_NG, 1, _BN)
    h = _init_h(sp3, species_embed)
    agg = _sc_msg(h.reshape(NCH * N, CHUNK), senders3, receivers3, rw0, unitT)
    rw1 = _edge_prep2(vt, radial_w1)
    h = _node_upd(agg, h, sp3, msg_w0, self_w0, species_b0, ln_s0.reshape(1, D))
    agg = _sc_msg(h.reshape(NCH * N, CHUNK), senders3, receivers3, rw1, unitT)
    h = _node_upd(agg, h, sp3, msg_w1, self_w1, species_b1, ln_s1.reshape(1, D))
    gid3 = graph_ids.reshape(_NG, 1, _BN)
    return _readout(h, gid3, readout_w, head_w1, head_w2,
                    out_scale.reshape(1), out_shift.reshape(1))
